# Initial kernel scaffold; baseline (speedup 1.0000x reference)
import jax
import jax.numpy as jnp
from jax.experimental import pallas as pl


def kernel(x_a, x_b, edge_index_aa, adj_val_aa, edge_index_ab, adj_val_ab, edge_index_ba, adj_val_ba, edge_index_bb, adj_val_bb, Wrel_a_a, Wrel_a_b, wself_a, bias_a, wcat_a, wquery_a, wkeys_a, watt_a, Wrel_b_a, Wrel_b_b, wself_b, bias_b, wcat_b, wquery_b, wkeys_b, watt_b):
    raise NotImplementedError("write your pallas kernel here")


# speedup vs baseline: 3.5551x; 3.5551x over previous
import jax
import jax.numpy as jnp
from jax import lax
from jax.experimental import pallas as pl
from jax.experimental.pallas import tpu as pltpu
from jax.experimental.pallas import tpu_sc as plsc

_N = 10000
_D = 128
_ATT = 64
_E = 320000
_L = 16
_NS = 16
_CPT = 157
_EP = _CPT * _NS * 128
_ZR = 624


def _sc_spmm(xa, xb, edges):
    mesh = plsc.VectorSubcoreMesh(core_axis_name="c", subcore_axis_name="s")
    zr = jnp.zeros((_ZR, _D), jnp.float32)
    out_type = tuple(jax.ShapeDtypeStruct((_N, _D), jnp.float32) for _ in range(4))
    scratch = [
        pltpu.VMEM_SHARED((_N, _D), jnp.float32),
        pltpu.VMEM((128, _D), jnp.float32),
        pltpu.VMEM((128,), jnp.int32),
        pltpu.VMEM((128,), jnp.int32),
        pltpu.VMEM((128,), jnp.float32),
        pltpu.SemaphoreType.DMA,
    ]

    def body(xa_h, xb_h, z_h,
             r0, c0, v0, r1, c1, v1, r2, c2, v2, r3, c3, v3,
             o0, o1, o2, o3,
             acc, gbuf, colv, rowv, valv, sem):
        cid = lax.axis_index("c")
        sid = lax.axis_index("s")

        def do_rel(x_h, rows_h, cols_h, vals_h, out_h):
            pltpu.sync_copy(z_h, acc.at[pl.ds(sid * _ZR, _ZR)])

            @pl.when(sid == 0)
            def _():
                pltpu.sync_copy(z_h.at[pl.ds(0, 16)],
                                acc.at[pl.ds(_NS * _ZR, 16)])

            plsc.subcore_barrier()

            def step(k, carry):
                base = (sid * _CPT + k) * 128
                pltpu.sync_copy(cols_h.at[pl.ds(base, 128)], colv)
                pltpu.sync_copy(rows_h.at[pl.ds(base, 128)], rowv)
                pltpu.sync_copy(vals_h.at[pl.ds(base, 128)], valv)
                pltpu.async_copy(x_h.at[colv], gbuf, sem).wait()

                def grp(i, c2_):
                    off = pl.multiple_of(i * _L, _L)
                    v16 = valv[pl.ds(off, _L)]
                    for e in range(_L):
                        row = off + e
                        sv = v16[e]
                        for f in range(_D // _L):
                            sl = pl.ds(f * _L, _L)
                            gbuf[row, sl] = gbuf[row, sl] * sv
                    return c2_

                lax.fori_loop(0, 128 // _L, grp, 0)
                pltpu.sync_copy(gbuf, acc.at[rowv], add=True)
                return carry

            lax.fori_loop(0, _CPT, step, 0)
            plsc.subcore_barrier()
            pltpu.sync_copy(acc.at[pl.ds(sid * _ZR, _ZR)],
                            out_h.at[pl.ds(sid * _ZR, _ZR)])

            @pl.when(sid == 0)
            def _():
                pltpu.sync_copy(acc.at[pl.ds(_NS * _ZR, 16)],
                                out_h.at[pl.ds(_NS * _ZR, 16)])

            plsc.subcore_barrier()

        @pl.when(cid == 0)
        def _():
            do_rel(xa_h, r0, c0, v0, o0)
            do_rel(xb_h, r1, c1, v1, o1)

        @pl.when(cid == 1)
        def _():
            do_rel(xa_h, r2, c2, v2, o2)
            do_rel(xb_h, r3, c3, v3, o3)

    f = pl.kernel(body, out_type=out_type, mesh=mesh, scratch_types=scratch)
    (r0, c0, v0), (r1, c1, v1), (r2, c2, v2), (r3, c3, v3) = edges
    return f(xa, xb, zr, r0, c0, v0, r1, c1, v1, r2, c2, v2, r3, c3, v3)


def _elu(v):
    return jnp.where(v > 0, v, jnp.exp(v) - 1.0)


def _dst_block(x, s1, s2, w1, w2, wself, bias, wcat, wq, wk, wt):
    dot = lambda a, b: jnp.dot(a, b, preferred_element_type=jnp.float32)
    self_ft = dot(x, wself)
    nb1 = dot(s1, w1)
    nb2 = dot(s2, w2)
    q = dot(self_ft, wq)
    k1 = dot(nb1, wk)
    k2 = dot(nb2, wk)
    qs = dot(q, wt[_ATT:, :])
    e1 = _elu(dot(k1, wt[:_ATT, :]) + qs)
    e2 = _elu(dot(k2, wt[:_ATT, :]) + qs)
    m = jnp.maximum(e1, e2)
    x1 = jnp.exp(e1 - m)
    x2 = jnp.exp(e2 - m)
    inv = 1.0 / (x1 + x2)
    agg = nb1 * (x1 * inv) + nb2 * (x2 * inv)
    return dot(agg, wcat[:_D, :]) + dot(self_ft, wcat[_D:, :]) + bias


def _tc_post(xa, xb, s0, s1, s2, s3,
             Waa, Wab, wsa, ba, wca, wqa, wka, wta,
             Wba, Wbb, wsb, bb, wcb, wqb, wkb, wtb):
    B = 2000
    grid = (_N // B,)

    def row():
        return pl.BlockSpec((B, _D), lambda i: (i, 0))

    def full(a):
        nd = a.ndim
        return pl.BlockSpec(a.shape, lambda i, _nd=nd: (0,) * _nd)

    def tc_body(xa_r, xb_r, s0_r, s1_r, s2_r, s3_r,
                Waa_r, Wab_r, wsa_r, ba_r, wca_r, wqa_r, wka_r, wta_r,
                Wba_r, Wbb_r, wsb_r, bb_r, wcb_r, wqb_r, wkb_r, wtb_r,
                oa_r, ob_r):
        oa_r[...] = _dst_block(xa_r[...], s0_r[...], s1_r[...],
                               Waa_r[...], Wab_r[...], wsa_r[...], ba_r[...],
                               wca_r[...], wqa_r[...], wka_r[...], wta_r[...])
        ob_r[...] = _dst_block(xb_r[...], s2_r[...], s3_r[...],
                               Wba_r[...], Wbb_r[...], wsb_r[...], bb_r[...],
                               wcb_r[...], wqb_r[...], wkb_r[...], wtb_r[...])

    args = (xa, xb, s0, s1, s2, s3, Waa, Wab, wsa, ba, wca, wqa, wka, wta,
            Wba, Wbb, wsb, bb, wcb, wqb, wkb, wtb)
    in_specs = [row()] * 6 + [full(a) for a in args[6:]]
    out = pl.pallas_call(
        tc_body,
        grid=grid,
        in_specs=in_specs,
        out_specs=[row(), row()],
        out_shape=[jax.ShapeDtypeStruct((_N, _D), jnp.float32)] * 2,
    )(*args)
    return out[0], out[1]


def kernel(x_a, x_b, edge_index_aa, adj_val_aa, edge_index_ab, adj_val_ab,
           edge_index_ba, adj_val_ba, edge_index_bb, adj_val_bb,
           Wrel_a_a, Wrel_a_b, wself_a, bias_a, wcat_a, wquery_a, wkeys_a,
           watt_a, Wrel_b_a, Wrel_b_b, wself_b, bias_b, wcat_b, wquery_b,
           wkeys_b, watt_b):
    pad = _EP - _E

    def prep(ei, v):
        return (jnp.pad(ei[0], (0, pad)), jnp.pad(ei[1], (0, pad)),
                jnp.pad(v, (0, pad)))

    edges = [prep(edge_index_aa, adj_val_aa), prep(edge_index_ab, adj_val_ab),
             prep(edge_index_ba, adj_val_ba), prep(edge_index_bb, adj_val_bb)]
    s0, s1, s2, s3 = _sc_spmm(x_a, x_b, edges)
    return _tc_post(x_a, x_b, s0, s1, s2, s3,
                    Wrel_a_a, Wrel_a_b, wself_a, bias_a, wcat_a, wquery_a,
                    wkeys_a, watt_a, Wrel_b_a, Wrel_b_b, wself_b, bias_b,
                    wcat_b, wquery_b, wkeys_b, watt_b)
